# baseline (speedup 1.0000x reference)
import functools

import jax
import jax.numpy as jnp
from jax import lax
from jax.experimental import pallas as pl
from jax.experimental.pallas import tpu as pltpu
from jax.experimental.pallas import tpu_sc as plsc

_WINS = (7, 15, 31)
_MIND, _MAXD = 0.05, 0.95
_GAMMA = 0.8
_B, _N, _NB, _H, _W = 4, 300, 16, 480, 640
_P = _B * _N
_G = _P // 16
_FETCHW = 48
_BIG = 1e9
_HUGE = 1e30

_NC, _NS = 2, 16
_NW = _NC * _NS


def _task_plan(wid):
    u = wid - 11
    heavy = wid < 11
    base31 = jnp.where(heavy, 3 * wid, 33 + 2 * u)
    cnt31 = jnp.where(heavy, 3, 2)
    base15 = jnp.where(u < 12, 4 * u, 48 + 3 * (u - 12))
    cnt15 = jnp.where(heavy, 0, jnp.where(u < 12, 4, 3))
    base7 = jnp.where(u < 9, 3 * u, 27 + 4 * (u - 9))
    cnt7 = jnp.where(heavy, 0, jnp.where(u < 9, 3, 4))
    return {31: (base31, cnt31, 3), 15: (base15, cnt15, 4),
            7: (base7, cnt7, 4)}


def _sc_loss_body(depth_hbm, meta7, meta15, meta31, bins7, bins15, bins31,
                  out_hbm, meta_v0, meta_v1, bins_v0, bins_v1,
                  patch_v0, patch_v1, acc_v,
                  semm0, semm1, semr0, semr1):
    wid = lax.axis_index("s") * _NC + lax.axis_index("c")
    lane = lax.iota(jnp.int32, 16)
    zero = jnp.zeros((16,), jnp.float32)
    for r in range(9):
        acc_v[r] = zero

    meta_bufs = (meta_v0, meta_v1)
    bins_bufs = (bins_v0, bins_v1)
    patch_bufs = (patch_v0, patch_v1)
    semm = (semm0, semm1)
    semr = (semr0, semr1)
    plan = _task_plan(wid)

    def tree_min(vs):
        while len(vs) > 1:
            vs = [jnp.minimum(vs[i], vs[i + 1])
                  for i in range(0, len(vs) - 1, 2)] + (
                      [vs[-1]] if len(vs) % 2 else [])
        return vs[0]

    def run_window(win, meta_hbm, bins_hbm, row_a, row_b, row_v):
        base, cnt, maxj = plan[win]

        def meta_copy(jj, slot):
            return pltpu.make_async_copy(
                meta_hbm.at[:, pl.ds((base + jj) * 16, 16)],
                meta_bufs[slot], semm[slot])

        def bins_copy(jj, slot):
            return pltpu.make_async_copy(
                bins_hbm.at[:, :, pl.ds((base + jj) * 16, 16)],
                bins_bufs[slot], semr[slot])

        def fire_patches(slot):
            r0_row = meta_bufs[slot][0]
            c0_row = meta_bufs[slot][1]
            for q in range(16):
                r0 = r0_row[q]
                c0 = pl.multiple_of(c0_row[q], 16)
                pltpu.async_copy(
                    depth_hbm.at[pl.ds(r0, win), pl.ds(c0, _FETCHW)],
                    patch_bufs[slot].at[q, pl.ds(0, win)], semr[slot])

        def wait_patches(slot):
            for q in range(16):
                pltpu.make_async_copy(
                    depth_hbm.at[pl.ds(0, win), pl.ds(0, _FETCHW)],
                    patch_bufs[slot].at[q, pl.ds(0, win)],
                    semr[slot]).wait()

        @pl.when(cnt > 0)
        def _():
            meta_copy(0, 0).start()
            meta_copy(0, 0).wait()
            fire_patches(0)
            bins_copy(0, 0).start()

            @pl.when(cnt > 1)
            def _():
                meta_copy(1, 1).start()
                bins_copy(1, 1).start()

        for j in range(maxj):
            slot = j % 2
            nslot = 1 - slot

            @pl.when(j < cnt)
            def _():
                dx_v = meta_bufs[slot][2]

                @pl.when(j + 1 < cnt)
                def _():
                    meta_copy(j + 1, nslot).wait()
                    fire_patches(nslot)

                @pl.when(j + 2 < cnt)
                def _():
                    meta_copy(j + 2, slot).start()

                bins_copy(j, slot).wait()
                wait_patches(slot)

                bins_v = bins_bufs[slot]
                patch_v = patch_bufs[slot]

                def col_body(r, c, carry):
                    minxa, minxb, sumya, sumyb, cntf = carry
                    idx_r = jnp.full((16,), 0, jnp.int32) + r
                    idx_c = dx_v + c
                    d = plsc.load_gather(patch_v, [lane, idx_r, idx_c])
                    m = (d > _MIND) & (d < _MAXD)
                    dp = jnp.where(m, d, _BIG)
                    mf = jnp.where(m, 1.0, 0.0)
                    new_minxa = []
                    mya = None
                    for i in range(_NB):
                        v = dp - bins_v[0, i]
                        v = v * v
                        new_minxa.append(jnp.minimum(minxa[i], v))
                        mya = v if mya is None else jnp.minimum(mya, v)
                    new_minxb = []
                    myb = None
                    for i in range(_NB):
                        v = dp - bins_v[1, i]
                        v = v * v
                        new_minxb.append(jnp.minimum(minxb[i], v))
                        myb = v if myb is None else jnp.minimum(myb, v)
                    sumya = sumya + mya * mf
                    sumyb = sumyb + myb * mf
                    cntf = cntf + mf
                    return (tuple(new_minxa), tuple(new_minxb),
                            sumya, sumyb, cntf)

                def row_body(r, carry):
                    return plsc.parallel_loop(
                        0, win, 1, unroll=1, carry=carry)(
                            lambda c, cy: col_body(r, c, cy))

                huge = jnp.full((16,), _HUGE, jnp.float32)
                carry0 = (tuple(huge for _ in range(_NB)),
                          tuple(huge for _ in range(_NB)),
                          zero, zero, zero)
                minxa, minxb, sumya, sumyb, cntf = lax.fori_loop(
                    0, win, row_body, carry0)

                cham_xa = functools.reduce(jnp.add, minxa) * (1.0 / _NB)
                cham_xb = functools.reduce(jnp.add, minxb) * (1.0 / _NB)
                safec = jnp.maximum(cntf, 1.0)
                pera = cham_xa + sumya / safec
                perb = cham_xb + sumyb / safec
                validf = jnp.where(cntf + cntf > float(win * win), 1.0, 0.0)
                acc_v[row_a] = acc_v[row_a] + pera * validf
                acc_v[row_b] = acc_v[row_b] + perb * validf
                acc_v[row_v] = acc_v[row_v] + validf

                @pl.when(j + 2 < cnt)
                def _():
                    bins_copy(j + 2, slot).start()

    run_window(7, meta7, bins7, 0, 3, 6)
    run_window(15, meta15, bins15, 1, 4, 7)
    run_window(31, meta31, bins31, 2, 5, 8)

    pltpu.sync_copy(acc_v, out_hbm.at[wid])


@functools.cache
def _build_sc_loss():
    mesh = plsc.VectorSubcoreMesh(core_axis_name="c", subcore_axis_name="s")
    return functools.partial(
        pl.kernel,
        mesh=mesh,
        out_type=jax.ShapeDtypeStruct((_NW, 9, 16), jnp.float32),
        scratch_types=[
            pltpu.VMEM((3, 16), jnp.int32),
            pltpu.VMEM((3, 16), jnp.int32),
            pltpu.VMEM((2, _NB, 16), jnp.float32),
            pltpu.VMEM((2, _NB, 16), jnp.float32),
            pltpu.VMEM((16, 31, _FETCHW), jnp.float32),
            pltpu.VMEM((16, 31, _FETCHW), jnp.float32),
            pltpu.VMEM((9, 16), jnp.float32),
            pltpu.SemaphoreType.DMA,
            pltpu.SemaphoreType.DMA,
            pltpu.SemaphoreType.DMA,
            pltpu.SemaphoreType.DMA,
        ],
        compiler_params=pltpu.CompilerParams(
            use_tc_tiling_on_sc=False, needs_layout_passes=False),
    )(_sc_loss_body)


def _prep(coords, bins_a, bins_b, win):
    k = win // 2
    x = coords[..., 0].reshape(_P).astype(jnp.int32)
    y = coords[..., 1].reshape(_P).astype(jnp.int32)
    bidx = jnp.arange(_P, dtype=jnp.int32) // _N
    x0 = x - k
    c0 = jnp.minimum(x0 & ~15, _W - _FETCHW)
    dx = x0 - c0
    r0 = bidx * _H + (y - k)
    meta = jnp.stack([r0, c0, dx]).astype(jnp.int32)
    binsT = jnp.stack([bins_a.reshape(_P, _NB).T,
                       bins_b.reshape(_P, _NB).T])
    return meta, binsT


def kernel(target_depth_maps, coords_w7, coords_w15, coords_w31,
           bins_s2_w7, bins_s2_w15, bins_s2_w31,
           bins_s1_w7, bins_s1_w15, bins_s1_w31):
    depth2d = target_depth_maps.reshape(_B * _H, _W)
    meta7, binsT7 = _prep(coords_w7, bins_s2_w7, bins_s1_w7, 7)
    meta15, binsT15 = _prep(coords_w15, bins_s2_w15, bins_s1_w15, 15)
    meta31, binsT31 = _prep(coords_w31, bins_s2_w31, bins_s1_w31, 31)

    out = _build_sc_loss()(depth2d, meta7, meta15, meta31,
                           binsT7, binsT15, binsT31)

    sums = out.sum(axis=(0, 2))
    cnts = jnp.maximum(sums[6:9], 1.0)
    la = sums[0:3] / cnts
    lb = sums[3:6] / cnts
    wc = jnp.array([1.0, _GAMMA, _GAMMA * _GAMMA], jnp.float32)
    sw = 1.0 + _GAMMA + _GAMMA * _GAMMA
    total = 1.0 * jnp.dot(la, wc) / sw + 0.5 * jnp.dot(lb, wc) / sw
    return total.astype(jnp.float32)


# speedup vs baseline: 1.0015x; 1.0015x over previous
import functools

import jax
import jax.numpy as jnp
from jax import lax
from jax.experimental import pallas as pl
from jax.experimental.pallas import tpu as pltpu
from jax.experimental.pallas import tpu_sc as plsc

_WINS = (7, 15, 31)
_MIND, _MAXD = 0.05, 0.95
_GAMMA = 0.8
_B, _N, _NB, _H, _W = 4, 300, 16, 480, 640
_P = _B * _N
_G = _P // 16
_FETCHW = 48
_BIG = 1e9
_HUGE = 1e30

_NC, _NS = 2, 16
_NW = _NC * _NS


def _task_plan(wid):
    u = wid - 11
    heavy = wid < 11
    base31 = jnp.where(heavy, 3 * wid, 33 + 2 * u)
    cnt31 = jnp.where(heavy, 3, 2)
    base15 = jnp.where(u < 12, 4 * u, 48 + 3 * (u - 12))
    cnt15 = jnp.where(heavy, 0, jnp.where(u < 12, 4, 3))
    base7 = jnp.where(u < 9, 3 * u, 27 + 4 * (u - 9))
    cnt7 = jnp.where(heavy, 0, jnp.where(u < 9, 3, 4))
    return {31: (base31, cnt31, 3), 15: (base15, cnt15, 4),
            7: (base7, cnt7, 4)}


def _sc_loss_body(depth_hbm, meta7, meta15, meta31, bins7, bins15, bins31,
                  out_hbm, meta_v0, meta_v1, bins_v0, bins_v1,
                  patch_v0, patch_v1, acc_v,
                  semm0, semm1, semr0, semr1):
    wid = lax.axis_index("s") * _NC + lax.axis_index("c")
    lane = lax.iota(jnp.int32, 16)
    zero = jnp.zeros((16,), jnp.float32)
    for r in range(9):
        acc_v[r] = zero

    meta_bufs = (meta_v0, meta_v1)
    bins_bufs = (bins_v0, bins_v1)
    patch_bufs = (patch_v0, patch_v1)
    semm = (semm0, semm1)
    semr = (semr0, semr1)
    plan = _task_plan(wid)

    def tree_min(vs):
        while len(vs) > 1:
            vs = [jnp.minimum(vs[i], vs[i + 1])
                  for i in range(0, len(vs) - 1, 2)] + (
                      [vs[-1]] if len(vs) % 2 else [])
        return vs[0]

    def run_window(win, meta_hbm, bins_hbm, row_a, row_b, row_v):
        base, cnt, maxj = plan[win]

        def meta_copy(jj, slot):
            return pltpu.make_async_copy(
                meta_hbm.at[:, pl.ds((base + jj) * 16, 16)],
                meta_bufs[slot], semm[slot])

        def bins_copy(jj, slot):
            return pltpu.make_async_copy(
                bins_hbm.at[:, :, pl.ds((base + jj) * 16, 16)],
                bins_bufs[slot], semr[slot])

        def fire_patches(slot):
            r0_row = meta_bufs[slot][0]
            c0_row = meta_bufs[slot][1]
            for q in range(16):
                r0 = r0_row[q]
                c0 = pl.multiple_of(c0_row[q], 16)
                pltpu.async_copy(
                    depth_hbm.at[pl.ds(r0, win), pl.ds(c0, _FETCHW)],
                    patch_bufs[slot].at[q, pl.ds(0, win)], semr[slot])

        def wait_patches(slot):
            for q in range(16):
                pltpu.make_async_copy(
                    depth_hbm.at[pl.ds(0, win), pl.ds(0, _FETCHW)],
                    patch_bufs[slot].at[q, pl.ds(0, win)],
                    semr[slot]).wait()

        @pl.when(cnt > 0)
        def _():
            meta_copy(0, 0).start()
            meta_copy(0, 0).wait()
            fire_patches(0)
            bins_copy(0, 0).start()

            @pl.when(cnt > 1)
            def _():
                meta_copy(1, 1).start()
                bins_copy(1, 1).start()

        for j in range(maxj):
            slot = j % 2
            nslot = 1 - slot

            @pl.when(j < cnt)
            def _():
                dx_v = meta_bufs[slot][2]

                @pl.when(j + 1 < cnt)
                def _():
                    meta_copy(j + 1, nslot).wait()
                    fire_patches(nslot)

                @pl.when(j + 2 < cnt)
                def _():
                    meta_copy(j + 2, slot).start()

                bins_copy(j, slot).wait()
                wait_patches(slot)

                bins_v = bins_bufs[slot]
                patch_v = patch_bufs[slot]

                def col_body(r, c, carry):
                    minxa, minxb, sumya, sumyb, cntf = carry
                    idx_r = jnp.full((16,), 0, jnp.int32) + r
                    idx_c = dx_v + c
                    d = plsc.load_gather(patch_v, [lane, idx_r, idx_c])
                    m = (d > _MIND) & (d < _MAXD)
                    dp = jnp.where(m, d, _BIG)
                    mf = jnp.where(m, 1.0, 0.0)
                    new_minxa = []
                    mya = None
                    for i in range(_NB):
                        v = dp - bins_v[0, i]
                        v = v * v
                        new_minxa.append(jnp.minimum(minxa[i], v))
                        mya = v if mya is None else jnp.minimum(mya, v)
                    new_minxb = []
                    myb = None
                    for i in range(_NB):
                        v = dp - bins_v[1, i]
                        v = v * v
                        new_minxb.append(jnp.minimum(minxb[i], v))
                        myb = v if myb is None else jnp.minimum(myb, v)
                    sumya = sumya + mya * mf
                    sumyb = sumyb + myb * mf
                    cntf = cntf + mf
                    return (tuple(new_minxa), tuple(new_minxb),
                            sumya, sumyb, cntf)

                def row_body(r, carry):
                    return lax.fori_loop(
                        0, win, lambda c, cy: col_body(r, c, cy), carry)

                huge = jnp.full((16,), _HUGE, jnp.float32)
                carry0 = (tuple(huge for _ in range(_NB)),
                          tuple(huge for _ in range(_NB)),
                          zero, zero, zero)
                minxa, minxb, sumya, sumyb, cntf = lax.fori_loop(
                    0, win, row_body, carry0)

                cham_xa = functools.reduce(jnp.add, minxa) * (1.0 / _NB)
                cham_xb = functools.reduce(jnp.add, minxb) * (1.0 / _NB)
                safec = jnp.maximum(cntf, 1.0)
                pera = cham_xa + sumya / safec
                perb = cham_xb + sumyb / safec
                validf = jnp.where(cntf + cntf > float(win * win), 1.0, 0.0)
                acc_v[row_a] = acc_v[row_a] + pera * validf
                acc_v[row_b] = acc_v[row_b] + perb * validf
                acc_v[row_v] = acc_v[row_v] + validf

                @pl.when(j + 2 < cnt)
                def _():
                    bins_copy(j + 2, slot).start()

    run_window(7, meta7, bins7, 0, 3, 6)
    run_window(15, meta15, bins15, 1, 4, 7)
    run_window(31, meta31, bins31, 2, 5, 8)

    pltpu.sync_copy(acc_v, out_hbm.at[wid])


@functools.cache
def _build_sc_loss():
    mesh = plsc.VectorSubcoreMesh(core_axis_name="c", subcore_axis_name="s")
    return functools.partial(
        pl.kernel,
        mesh=mesh,
        out_type=jax.ShapeDtypeStruct((_NW, 9, 16), jnp.float32),
        scratch_types=[
            pltpu.VMEM((3, 16), jnp.int32),
            pltpu.VMEM((3, 16), jnp.int32),
            pltpu.VMEM((2, _NB, 16), jnp.float32),
            pltpu.VMEM((2, _NB, 16), jnp.float32),
            pltpu.VMEM((16, 31, _FETCHW), jnp.float32),
            pltpu.VMEM((16, 31, _FETCHW), jnp.float32),
            pltpu.VMEM((9, 16), jnp.float32),
            pltpu.SemaphoreType.DMA,
            pltpu.SemaphoreType.DMA,
            pltpu.SemaphoreType.DMA,
            pltpu.SemaphoreType.DMA,
        ],
        compiler_params=pltpu.CompilerParams(
            use_tc_tiling_on_sc=False, needs_layout_passes=False),
    )(_sc_loss_body)


def _prep(coords, bins_a, bins_b, win):
    k = win // 2
    x = coords[..., 0].reshape(_P).astype(jnp.int32)
    y = coords[..., 1].reshape(_P).astype(jnp.int32)
    bidx = jnp.arange(_P, dtype=jnp.int32) // _N
    x0 = x - k
    c0 = jnp.minimum(x0 & ~15, _W - _FETCHW)
    dx = x0 - c0
    r0 = bidx * _H + (y - k)
    meta = jnp.stack([r0, c0, dx]).astype(jnp.int32)
    binsT = jnp.stack([bins_a.reshape(_P, _NB).T,
                       bins_b.reshape(_P, _NB).T])
    return meta, binsT


def kernel(target_depth_maps, coords_w7, coords_w15, coords_w31,
           bins_s2_w7, bins_s2_w15, bins_s2_w31,
           bins_s1_w7, bins_s1_w15, bins_s1_w31):
    depth2d = target_depth_maps.reshape(_B * _H, _W)
    meta7, binsT7 = _prep(coords_w7, bins_s2_w7, bins_s1_w7, 7)
    meta15, binsT15 = _prep(coords_w15, bins_s2_w15, bins_s1_w15, 15)
    meta31, binsT31 = _prep(coords_w31, bins_s2_w31, bins_s1_w31, 31)

    out = _build_sc_loss()(depth2d, meta7, meta15, meta31,
                           binsT7, binsT15, binsT31)

    sums = out.sum(axis=(0, 2))
    cnts = jnp.maximum(sums[6:9], 1.0)
    la = sums[0:3] / cnts
    lb = sums[3:6] / cnts
    wc = jnp.array([1.0, _GAMMA, _GAMMA * _GAMMA], jnp.float32)
    sw = 1.0 + _GAMMA + _GAMMA * _GAMMA
    total = 1.0 * jnp.dot(la, wc) / sw + 0.5 * jnp.dot(lb, wc) / sw
    return total.astype(jnp.float32)
